# baseline (speedup 1.0000x reference)
import functools

import jax
import jax.numpy as jnp
from jax import lax
from jax.experimental import pallas as pl
from jax.experimental.pallas import tpu as pltpu
from jax.experimental.pallas import tpu_sc as plsc

NN = 10000
EE = 320000
FF = 128
GG = 128
NC, NS = 2, 16
NW = NC * NS
CHUNK = 80
EPW = EE // NW
EPWP = 10240
NCHUNK = EPWP // CHUNK
IBLK = 64
NP = 10240
RPT = NP // NS
ZCH = 40
DEGW = 128

_f32 = jnp.float32


def _fill_rows(buf, nrows, width, value):
    vec = jnp.full((16,), value, _f32)

    def body(r, c):
        for k in range(width // 16):
            buf[r, pl.ds(k * 16, 16)] = vec
        return c

    lax.fori_loop(0, nrows, body, 0)


def _deg_body(dst3, deg_out, idx_v, ones_v, zbuf, deg_sh):
    cid = lax.axis_index("c")
    sid = lax.axis_index("s")
    wid = cid * NS + sid
    pltpu.sync_copy(dst3.at[wid], idx_v)
    _fill_rows(ones_v, CHUNK, DEGW, 1.0)
    _fill_rows(zbuf, ZCH, DEGW, 0.0)
    for t in range(RPT // ZCH):
        pltpu.sync_copy(zbuf, deg_sh.at[pl.ds(sid * RPT + t * ZCH, ZCH)])
    plsc.subcore_barrier()

    def body(j, c):
        pltpu.sync_copy(ones_v, deg_sh.at[idx_v.at[j]], add=True)
        return c

    lax.fori_loop(0, NCHUNK, body, 0)
    plsc.subcore_barrier()
    pltpu.sync_copy(deg_sh.at[pl.ds(sid * RPT, RPT)],
                    deg_out.at[cid, pl.ds(sid * RPT, RPT)])


_sc_deg = functools.partial(
    pl.kernel,
    out_type=jax.ShapeDtypeStruct((NC, NP, DEGW), _f32),
    mesh=plsc.VectorSubcoreMesh(core_axis_name="c", subcore_axis_name="s",
                                num_cores=NC, num_subcores=NS),
    scratch_types=[
        pltpu.VMEM((NCHUNK, CHUNK), jnp.int32),
        pltpu.VMEM((CHUNK, DEGW), _f32),
        pltpu.VMEM((ZCH, DEGW), _f32),
        pltpu.VMEM_SHARED((NP, DEGW), _f32),
    ],
)(_deg_body)


def _agg_body(y, src3, dst3, out, idx_s, idx_d, rows, zbuf, acc_sh, sem, sem2):
    cid = lax.axis_index("c")
    sid = lax.axis_index("s")
    wid = cid * NS + sid
    _fill_rows(zbuf, ZCH, FF, 0.0)
    for t in range(RPT // ZCH):
        pltpu.sync_copy(zbuf, acc_sh.at[pl.ds(sid * RPT + t * ZCH, ZCH)])
    plsc.subcore_barrier()

    for b in range(NCHUNK // IBLK):
        pltpu.sync_copy(src3.at[wid, pl.ds(b * IBLK, IBLK)], idx_s)
        pltpu.sync_copy(dst3.at[wid, pl.ds(b * IBLK, IBLK)], idx_d)

        def body(j, c):
            pltpu.async_copy(y.at[idx_s.at[j]], rows, sem).wait()
            pltpu.sync_copy(rows, acc_sh.at[idx_d.at[j]], add=True)
            return c

        lax.fori_loop(0, IBLK, body, 0)
    plsc.subcore_barrier()
    pltpu.sync_copy(acc_sh.at[pl.ds(sid * RPT, RPT)],
                    out.at[cid, pl.ds(sid * RPT, RPT)])


_sc_agg = functools.partial(
    pl.kernel,
    out_type=jax.ShapeDtypeStruct((NC, NP, FF), _f32),
    mesh=plsc.VectorSubcoreMesh(core_axis_name="c", subcore_axis_name="s",
                                num_cores=NC, num_subcores=NS),
    scratch_types=[
        pltpu.VMEM((IBLK, CHUNK), jnp.int32),
        pltpu.VMEM((IBLK, CHUNK), jnp.int32),
        pltpu.VMEM((CHUNK, FF), _f32),
        pltpu.VMEM((ZCH, FF), _f32),
        pltpu.VMEM_SHARED((NP, FF), _f32),
        pltpu.SemaphoreType.DMA,
        pltpu.SemaphoreType.DMA,
    ],
)(_agg_body)


def _pre_body(x_ref, w_ref, deg_ref, y_ref, dinv_ref):
    deg = deg_ref[0, :NN, 0:1] + deg_ref[1, :NN, 0:1] + 1.0
    dinv = lax.rsqrt(deg)
    dinv_b = jnp.broadcast_to(dinv, (NN, FF))
    dinv_ref[...] = dinv_b
    xw = jnp.dot(x_ref[...], w_ref[...], preferred_element_type=_f32)
    y_ref[...] = xw * dinv_b


def _tc_pre(x, w1, deg):
    return pl.pallas_call(
        _pre_body,
        out_shape=(jax.ShapeDtypeStruct((NN, FF), _f32),
                   jax.ShapeDtypeStruct((NN, FF), _f32)),
    )(x, w1, deg)


def _layer_tail(prev, y, agg0, agg1, dinv, b, g, be):
    conv = dinv * (agg0 + agg1 + y) + b
    t = prev + conv
    m = jnp.mean(t, axis=0, keepdims=True)
    v = jnp.mean((t - m) * (t - m), axis=0, keepdims=True)
    xh = (t - m) * lax.rsqrt(v + 1e-5) * g + be
    neg = jnp.where(xh > 0, 0.0, xh)
    return jnp.where(xh > 0, xh, jnp.exp(neg) - 1.0)


def _mid_body(prev_ref, y_ref, agg_ref, dinv_ref, b_ref, g_ref, be_ref,
              wn_ref, hn_ref, yn_ref):
    dinv = dinv_ref[...]
    hn = _layer_tail(prev_ref[...], y_ref[...], agg_ref[0, :NN], agg_ref[1, :NN],
                     dinv, b_ref[...], g_ref[...], be_ref[...])
    hn_ref[...] = hn
    yn_ref[...] = jnp.dot(hn, wn_ref[...], preferred_element_type=_f32) * dinv


def _tc_mid(prev, y, agg, dinv, b, g, be, wn):
    return pl.pallas_call(
        _mid_body,
        out_shape=(jax.ShapeDtypeStruct((NN, FF), _f32),
                   jax.ShapeDtypeStruct((NN, FF), _f32)),
    )(prev, y, agg, dinv, b, g, be, wn)


def _final_body(prev_ref, y_ref, agg_ref, dinv_ref, b_ref, g_ref, be_ref,
                batch_ref, wr_ref, br_ref, out_ref):
    hn = _layer_tail(prev_ref[...], y_ref[...], agg_ref[0, :NN], agg_ref[1, :NN],
                     dinv_ref[...], b_ref[...], g_ref[...], be_ref[...])
    cols = lax.broadcasted_iota(jnp.int32, (NN, GG), 1)
    oh = (batch_ref[...] == cols).astype(_f32)
    dn = (((0,), (0,)), ((), ()))
    sums = lax.dot_general(oh, hn, dn, preferred_element_type=_f32)
    ones = jnp.ones((NN, 8), _f32)
    counts = lax.dot_general(oh, ones, dn, preferred_element_type=_f32)[:, 0:1]
    pooled = sums / jnp.maximum(counts, 1.0)
    out_ref[...] = (jnp.dot(pooled, wr_ref[...], preferred_element_type=_f32)
                    + br_ref[...])


def _tc_final(prev, y, agg, dinv, b, g, be, batch2d, wr, br):
    return pl.pallas_call(
        _final_body,
        out_shape=jax.ShapeDtypeStruct((GG, 2), _f32),
    )(prev, y, agg, dinv, b, g, be, batch2d, wr, br)


def kernel(x, edge_index, batch, W1, b1, g1, be1, W2, b2, g2, be2,
           W3, b3, g3, be3, Wr, br):
    npad = EPWP - EPW
    src_pad = jnp.zeros((NW, npad), jnp.int32)
    dst_pad = jnp.broadcast_to(
        10048 + (jnp.arange(npad, dtype=jnp.int32) % 192), (NW, npad))
    src3 = jnp.concatenate([edge_index[0].reshape(NW, EPW), src_pad],
                           axis=1).reshape(NW, NCHUNK, CHUNK)
    dst3 = jnp.concatenate([edge_index[1].reshape(NW, EPW), dst_pad],
                           axis=1).reshape(NW, NCHUNK, CHUNK)
    batch2d = batch.reshape(NN, 1)
    b1r, g1r, be1r = b1.reshape(1, FF), g1.reshape(1, FF), be1.reshape(1, FF)
    b2r, g2r, be2r = b2.reshape(1, FF), g2.reshape(1, FF), be2.reshape(1, FF)
    b3r, g3r, be3r = b3.reshape(1, FF), g3.reshape(1, FF), be3.reshape(1, FF)
    brr = br.reshape(1, 2)
    zeros = jnp.zeros((NN, FF), _f32)

    deg = _sc_deg(dst3)
    y1, dinv = _tc_pre(x, W1, deg)
    agg1 = _sc_agg(y1, src3, dst3)
    h1, y2 = _tc_mid(zeros, y1, agg1, dinv, b1r, g1r, be1r, W2)
    agg2 = _sc_agg(y2, src3, dst3)
    h2, y3 = _tc_mid(x, y2, agg2, dinv, b2r, g2r, be2r, W3)
    agg3 = _sc_agg(y3, src3, dst3)
    out = _tc_final(h1, y3, agg3, dinv, b3r, g3r, be3r, batch2d, Wr, brr)
    return out


# speedup vs baseline: 1.0041x; 1.0041x over previous
import functools

import jax
import jax.numpy as jnp
from jax import lax
from jax.experimental import pallas as pl
from jax.experimental.pallas import tpu as pltpu
from jax.experimental.pallas import tpu_sc as plsc

NN = 10000
EE = 320000
FF = 128
GG = 128
NC, NS = 2, 16
NW = NC * NS
CHUNK = 80
EPW = EE // NW
EPWP = 10240
NCHUNK = EPWP // CHUNK
IBLK = 64
NP = 10240
RPT = NP // NS
ZCH = 40
DEGW = 128

_f32 = jnp.float32


def _fill_rows(buf, nrows, width, value):
    vec = jnp.full((16,), value, _f32)

    def body(r, c):
        for k in range(width // 16):
            buf[r, pl.ds(k * 16, 16)] = vec
        return c

    lax.fori_loop(0, nrows, body, 0)


def _deg_body(dst3, deg_out, idx_v, ones_v, zbuf, deg_sh):
    cid = lax.axis_index("c")
    sid = lax.axis_index("s")
    wid = cid * NS + sid
    pltpu.sync_copy(dst3.at[wid], idx_v)
    _fill_rows(ones_v, CHUNK, DEGW, 1.0)
    _fill_rows(zbuf, ZCH, DEGW, 0.0)
    for t in range(RPT // ZCH):
        pltpu.sync_copy(zbuf, deg_sh.at[pl.ds(sid * RPT + t * ZCH, ZCH)])
    plsc.subcore_barrier()

    def body(j, c):
        pltpu.sync_copy(ones_v, deg_sh.at[idx_v.at[j]], add=True)
        return c

    lax.fori_loop(0, NCHUNK, body, 0)
    plsc.subcore_barrier()
    pltpu.sync_copy(deg_sh.at[pl.ds(sid * RPT, RPT)],
                    deg_out.at[cid, pl.ds(sid * RPT, RPT)])


_sc_deg = functools.partial(
    pl.kernel,
    out_type=jax.ShapeDtypeStruct((NC, NP, DEGW), _f32),
    mesh=plsc.VectorSubcoreMesh(core_axis_name="c", subcore_axis_name="s",
                                num_cores=NC, num_subcores=NS),
    scratch_types=[
        pltpu.VMEM((NCHUNK, CHUNK), jnp.int32),
        pltpu.VMEM((CHUNK, DEGW), _f32),
        pltpu.VMEM((ZCH, DEGW), _f32),
        pltpu.VMEM_SHARED((NP, DEGW), _f32),
    ],
)(_deg_body)


def _agg_body(y, src3, dst3, out, idx_s, idx_d, rows, zbuf, acc_sh, sem, sem2):
    cid = lax.axis_index("c")
    sid = lax.axis_index("s")
    wid = cid * NS + sid
    _fill_rows(zbuf, ZCH, FF, 0.0)
    for t in range(RPT // ZCH):
        pltpu.sync_copy(zbuf, acc_sh.at[pl.ds(sid * RPT + t * ZCH, ZCH)])
    plsc.subcore_barrier()

    pltpu.sync_copy(src3.at[wid], idx_s)
    pltpu.sync_copy(dst3.at[wid], idx_d)

    def body(j, c):
        pltpu.async_copy(y.at[idx_s.at[j]], rows, sem).wait()
        pltpu.sync_copy(rows, acc_sh.at[idx_d.at[j]], add=True)
        return c

    lax.fori_loop(0, NCHUNK, body, 0)
    plsc.subcore_barrier()
    pltpu.sync_copy(acc_sh.at[pl.ds(sid * RPT, RPT)],
                    out.at[cid, pl.ds(sid * RPT, RPT)])


_sc_agg = functools.partial(
    pl.kernel,
    out_type=jax.ShapeDtypeStruct((NC, NP, FF), _f32),
    mesh=plsc.VectorSubcoreMesh(core_axis_name="c", subcore_axis_name="s",
                                num_cores=NC, num_subcores=NS),
    scratch_types=[
        pltpu.VMEM((NCHUNK, CHUNK), jnp.int32),
        pltpu.VMEM((NCHUNK, CHUNK), jnp.int32),
        pltpu.VMEM((CHUNK, FF), _f32),
        pltpu.VMEM((ZCH, FF), _f32),
        pltpu.VMEM_SHARED((NP, FF), _f32),
        pltpu.SemaphoreType.DMA,
        pltpu.SemaphoreType.DMA,
    ],
)(_agg_body)


def _pre_body(x_ref, w_ref, deg_ref, y_ref, dinv_ref):
    deg = deg_ref[0, :NN, 0:1] + deg_ref[1, :NN, 0:1] + 1.0
    dinv = lax.rsqrt(deg)
    dinv_b = jnp.broadcast_to(dinv, (NN, FF))
    dinv_ref[...] = dinv_b
    xw = jnp.dot(x_ref[...], w_ref[...], preferred_element_type=_f32)
    y_ref[...] = xw * dinv_b


def _tc_pre(x, w1, deg):
    return pl.pallas_call(
        _pre_body,
        out_shape=(jax.ShapeDtypeStruct((NN, FF), _f32),
                   jax.ShapeDtypeStruct((NN, FF), _f32)),
    )(x, w1, deg)


def _layer_tail(prev, y, agg0, agg1, dinv, b, g, be):
    conv = dinv * (agg0 + agg1 + y) + b
    t = prev + conv
    m = jnp.mean(t, axis=0, keepdims=True)
    v = jnp.mean((t - m) * (t - m), axis=0, keepdims=True)
    xh = (t - m) * lax.rsqrt(v + 1e-5) * g + be
    neg = jnp.where(xh > 0, 0.0, xh)
    return jnp.where(xh > 0, xh, jnp.exp(neg) - 1.0)


def _mid_body(prev_ref, y_ref, agg_ref, dinv_ref, b_ref, g_ref, be_ref,
              wn_ref, hn_ref, yn_ref):
    dinv = dinv_ref[...]
    hn = _layer_tail(prev_ref[...], y_ref[...], agg_ref[0, :NN], agg_ref[1, :NN],
                     dinv, b_ref[...], g_ref[...], be_ref[...])
    hn_ref[...] = hn
    yn_ref[...] = jnp.dot(hn, wn_ref[...], preferred_element_type=_f32) * dinv


def _tc_mid(prev, y, agg, dinv, b, g, be, wn):
    return pl.pallas_call(
        _mid_body,
        out_shape=(jax.ShapeDtypeStruct((NN, FF), _f32),
                   jax.ShapeDtypeStruct((NN, FF), _f32)),
    )(prev, y, agg, dinv, b, g, be, wn)


def _final_body(prev_ref, y_ref, agg_ref, dinv_ref, b_ref, g_ref, be_ref,
                batch_ref, wr_ref, br_ref, out_ref):
    hn = _layer_tail(prev_ref[...], y_ref[...], agg_ref[0, :NN], agg_ref[1, :NN],
                     dinv_ref[...], b_ref[...], g_ref[...], be_ref[...])
    cols = lax.broadcasted_iota(jnp.int32, (NN, GG), 1)
    oh = (batch_ref[...] == cols).astype(_f32)
    dn = (((0,), (0,)), ((), ()))
    sums = lax.dot_general(oh, hn, dn, preferred_element_type=_f32)
    ones = jnp.ones((NN, 8), _f32)
    counts = lax.dot_general(oh, ones, dn, preferred_element_type=_f32)[:, 0:1]
    pooled = sums / jnp.maximum(counts, 1.0)
    out_ref[...] = (jnp.dot(pooled, wr_ref[...], preferred_element_type=_f32)
                    + br_ref[...])


def _tc_final(prev, y, agg, dinv, b, g, be, batch2d, wr, br):
    return pl.pallas_call(
        _final_body,
        out_shape=jax.ShapeDtypeStruct((GG, 2), _f32),
    )(prev, y, agg, dinv, b, g, be, batch2d, wr, br)


def kernel(x, edge_index, batch, W1, b1, g1, be1, W2, b2, g2, be2,
           W3, b3, g3, be3, Wr, br):
    npad = EPWP - EPW
    src_pad = jnp.zeros((NW, npad), jnp.int32)
    dst_pad = jnp.broadcast_to(
        10048 + (jnp.arange(npad, dtype=jnp.int32) % 192), (NW, npad))
    src3 = jnp.concatenate([edge_index[0].reshape(NW, EPW), src_pad],
                           axis=1).reshape(NW, NCHUNK, CHUNK)
    dst3 = jnp.concatenate([edge_index[1].reshape(NW, EPW), dst_pad],
                           axis=1).reshape(NW, NCHUNK, CHUNK)
    batch2d = batch.reshape(NN, 1)
    b1r, g1r, be1r = b1.reshape(1, FF), g1.reshape(1, FF), be1.reshape(1, FF)
    b2r, g2r, be2r = b2.reshape(1, FF), g2.reshape(1, FF), be2.reshape(1, FF)
    b3r, g3r, be3r = b3.reshape(1, FF), g3.reshape(1, FF), be3.reshape(1, FF)
    brr = br.reshape(1, 2)
    zeros = jnp.zeros((NN, FF), _f32)

    deg = _sc_deg(dst3)
    y1, dinv = _tc_pre(x, W1, deg)
    agg1 = _sc_agg(y1, src3, dst3)
    h1, y2 = _tc_mid(zeros, y1, agg1, dinv, b1r, g1r, be1r, W2)
    agg2 = _sc_agg(y2, src3, dst3)
    h2, y3 = _tc_mid(x, y2, agg2, dinv, b2r, g2r, be2r, W3)
    agg3 = _sc_agg(y3, src3, dst3)
    out = _tc_final(h1, y3, agg3, dinv, b3r, g3r, be3r, batch2d, Wr, brr)
    return out


# speedup vs baseline: 2.0894x; 2.0808x over previous
import functools

import jax
import jax.numpy as jnp
from jax import lax
from jax.experimental import pallas as pl
from jax.experimental.pallas import tpu as pltpu
from jax.experimental.pallas import tpu_sc as plsc

NN = 10000
EE = 320000
FF = 128
GG = 128
NC, NS = 2, 16
NW = NC * NS
CHUNK = 80
EPW = EE // NW
NCHUNK = EPW // CHUNK
NP = 10240
RPT = NP // NS
ZCH = 40
DEGW = 128

_f32 = jnp.float32


def _fill_rows(buf, nrows, width, value):
    vec = jnp.full((16,), value, _f32)

    def body(r, c):
        for k in range(width // 16):
            buf[r, pl.ds(k * 16, 16)] = vec
        return c

    lax.fori_loop(0, nrows, body, 0)


def _deg_body(dst3, deg_out, idx_v, ones_v, zbuf, deg_sh):
    cid = lax.axis_index("c")
    sid = lax.axis_index("s")
    wid = cid * NS + sid
    pltpu.sync_copy(dst3.at[wid], idx_v)
    _fill_rows(ones_v, CHUNK, DEGW, 1.0)
    _fill_rows(zbuf, ZCH, DEGW, 0.0)
    for t in range(RPT // ZCH):
        pltpu.sync_copy(zbuf, deg_sh.at[pl.ds(sid * RPT + t * ZCH, ZCH)])
    plsc.subcore_barrier()

    def body(j, c):
        pltpu.sync_copy(ones_v, deg_sh.at[idx_v.at[j]], add=True)
        return c

    lax.fori_loop(0, NCHUNK, body, 0)
    plsc.subcore_barrier()
    pltpu.sync_copy(deg_sh.at[pl.ds(sid * RPT, RPT)],
                    deg_out.at[cid, pl.ds(sid * RPT, RPT)])


_sc_deg = functools.partial(
    pl.kernel,
    out_type=jax.ShapeDtypeStruct((NC, NP, DEGW), _f32),
    mesh=plsc.VectorSubcoreMesh(core_axis_name="c", subcore_axis_name="s",
                                num_cores=NC, num_subcores=NS),
    scratch_types=[
        pltpu.VMEM((NCHUNK, CHUNK), jnp.int32),
        pltpu.VMEM((CHUNK, DEGW), _f32),
        pltpu.VMEM((ZCH, DEGW), _f32),
        pltpu.VMEM_SHARED((NP, DEGW), _f32),
    ],
)(_deg_body)


def _agg_body(y, src3, dst3, out, idx_s, idx_d, rows, zbuf, acc_sh, sem, sem2):
    cid = lax.axis_index("c")
    sid = lax.axis_index("s")
    wid = cid * NS + sid
    _fill_rows(zbuf, ZCH, FF, 0.0)
    for t in range(RPT // ZCH):
        pltpu.sync_copy(zbuf, acc_sh.at[pl.ds(sid * RPT + t * ZCH, ZCH)])
    plsc.subcore_barrier()

    pltpu.sync_copy(src3.at[wid], idx_s)
    pltpu.sync_copy(dst3.at[wid], idx_d)

    def body(j, c):
        pltpu.async_copy(y.at[idx_s.at[j]], rows, sem).wait()
        pltpu.sync_copy(rows, acc_sh.at[idx_d.at[j]], add=True)
        return c

    lax.fori_loop(0, NCHUNK, body, 0)
    plsc.subcore_barrier()
    pltpu.sync_copy(acc_sh.at[pl.ds(sid * RPT, RPT)],
                    out.at[cid, pl.ds(sid * RPT, RPT)])


_sc_agg = functools.partial(
    pl.kernel,
    out_type=jax.ShapeDtypeStruct((NC, NP, FF), _f32),
    mesh=plsc.VectorSubcoreMesh(core_axis_name="c", subcore_axis_name="s",
                                num_cores=NC, num_subcores=NS),
    scratch_types=[
        pltpu.VMEM((NCHUNK, CHUNK), jnp.int32),
        pltpu.VMEM((NCHUNK, CHUNK), jnp.int32),
        pltpu.VMEM((CHUNK, FF), _f32),
        pltpu.VMEM((ZCH, FF), _f32),
        pltpu.VMEM_SHARED((NP, FF), _f32),
        pltpu.SemaphoreType.DMA,
        pltpu.SemaphoreType.DMA,
    ],
)(_agg_body)


def _pre_body(x_ref, w_ref, deg_ref, y_ref, dinv_ref):
    deg = deg_ref[0, :NN, 0:1] + deg_ref[1, :NN, 0:1] + 1.0
    dinv = lax.rsqrt(deg)
    dinv_b = jnp.broadcast_to(dinv, (NN, FF))
    dinv_ref[...] = dinv_b
    xw = jnp.dot(x_ref[...], w_ref[...], preferred_element_type=_f32)
    y_ref[...] = xw * dinv_b


def _tc_pre(x, w1, deg):
    return pl.pallas_call(
        _pre_body,
        out_shape=(jax.ShapeDtypeStruct((NN, FF), _f32),
                   jax.ShapeDtypeStruct((NN, FF), _f32)),
    )(x, w1, deg)


def _layer_tail(prev, y, agg0, agg1, dinv, b, g, be):
    conv = dinv * (agg0 + agg1 + y) + b
    t = prev + conv
    m = jnp.mean(t, axis=0, keepdims=True)
    v = jnp.mean((t - m) * (t - m), axis=0, keepdims=True)
    xh = (t - m) * lax.rsqrt(v + 1e-5) * g + be
    neg = jnp.where(xh > 0, 0.0, xh)
    return jnp.where(xh > 0, xh, jnp.exp(neg) - 1.0)


def _mid_body(prev_ref, y_ref, agg_ref, dinv_ref, b_ref, g_ref, be_ref,
              wn_ref, hn_ref, yn_ref):
    dinv = dinv_ref[...]
    hn = _layer_tail(prev_ref[...], y_ref[...], agg_ref[0, :NN], agg_ref[1, :NN],
                     dinv, b_ref[...], g_ref[...], be_ref[...])
    hn_ref[...] = hn
    yn_ref[...] = jnp.dot(hn, wn_ref[...], preferred_element_type=_f32) * dinv


def _tc_mid(prev, y, agg, dinv, b, g, be, wn):
    return pl.pallas_call(
        _mid_body,
        out_shape=(jax.ShapeDtypeStruct((NN, FF), _f32),
                   jax.ShapeDtypeStruct((NN, FF), _f32)),
    )(prev, y, agg, dinv, b, g, be, wn)


def _final_body(prev_ref, y_ref, agg_ref, dinv_ref, b_ref, g_ref, be_ref,
                batch_ref, wr_ref, br_ref, out_ref):
    hn = _layer_tail(prev_ref[...], y_ref[...], agg_ref[0, :NN], agg_ref[1, :NN],
                     dinv_ref[...], b_ref[...], g_ref[...], be_ref[...])
    cols = lax.broadcasted_iota(jnp.int32, (NN, GG), 1)
    oh = (batch_ref[...] == cols).astype(_f32)
    dn = (((0,), (0,)), ((), ()))
    sums = lax.dot_general(oh, hn, dn, preferred_element_type=_f32)
    ones = jnp.ones((NN, 8), _f32)
    counts = lax.dot_general(oh, ones, dn, preferred_element_type=_f32)[:, 0:1]
    pooled = sums / jnp.maximum(counts, 1.0)
    out_ref[...] = (jnp.dot(pooled, wr_ref[...], preferred_element_type=_f32)
                    + br_ref[...])


def _tc_final(prev, y, agg, dinv, b, g, be, batch2d, wr, br):
    return pl.pallas_call(
        _final_body,
        out_shape=jax.ShapeDtypeStruct((GG, 2), _f32),
    )(prev, y, agg, dinv, b, g, be, batch2d, wr, br)


def kernel(x, edge_index, batch, W1, b1, g1, be1, W2, b2, g2, be2,
           W3, b3, g3, be3, Wr, br):
    src3 = edge_index[0].reshape(NW, NCHUNK, CHUNK)
    dst3 = edge_index[1].reshape(NW, NCHUNK, CHUNK)
    batch2d = batch.reshape(NN, 1)
    b1r, g1r, be1r = b1.reshape(1, FF), g1.reshape(1, FF), be1.reshape(1, FF)
    b2r, g2r, be2r = b2.reshape(1, FF), g2.reshape(1, FF), be2.reshape(1, FF)
    b3r, g3r, be3r = b3.reshape(1, FF), g3.reshape(1, FF), be3.reshape(1, FF)
    brr = br.reshape(1, 2)
    zeros = jnp.zeros((NN, FF), _f32)

    deg = _sc_deg(dst3)
    y1, dinv = _tc_pre(x, W1, deg)
    agg1 = _sc_agg(y1, src3, dst3)
    h1, y2 = _tc_mid(zeros, y1, agg1, dinv, b1r, g1r, be1r, W2)
    agg2 = _sc_agg(y2, src3, dst3)
    h2, y3 = _tc_mid(x, y2, agg2, dinv, b2r, g2r, be2r, W3)
    agg3 = _sc_agg(y3, src3, dst3)
    out = _tc_final(h1, y3, agg3, dinv, b3r, g3r, be3r, batch2d, Wr, brr)
    return out


# speedup vs baseline: 2.4993x; 1.1962x over previous
import functools

import jax
import jax.numpy as jnp
from jax import lax
from jax.experimental import pallas as pl
from jax.experimental.pallas import tpu as pltpu
from jax.experimental.pallas import tpu_sc as plsc

NN = 10000
EE = 320000
FF = 128
GG = 128
NC, NS = 2, 16
NW = NC * NS
CHUNK = 80
EPW = EE // NW
NCHUNK = EPW // CHUNK
NP = 10240
RPT = NP // NS
ZCH = 40
DEGW = 128

_f32 = jnp.float32


def _fill_rows(buf, nrows, width, value):
    vec = jnp.full((16,), value, _f32)

    def body(r, c):
        for k in range(width // 16):
            buf[r, pl.ds(k * 16, 16)] = vec
        return c

    lax.fori_loop(0, nrows, body, 0)


def _deg_body(dst3, deg_out, idx_v, ones_v, zbuf, deg_sh):
    cid = lax.axis_index("c")
    sid = lax.axis_index("s")
    wid = cid * NS + sid
    pltpu.sync_copy(dst3.at[wid], idx_v)
    _fill_rows(ones_v, CHUNK, DEGW, 1.0)
    _fill_rows(zbuf, ZCH, DEGW, 0.0)
    for t in range(RPT // ZCH):
        pltpu.sync_copy(zbuf, deg_sh.at[pl.ds(sid * RPT + t * ZCH, ZCH)])
    plsc.subcore_barrier()

    def body(j, c):
        pltpu.sync_copy(ones_v, deg_sh.at[idx_v.at[j]], add=True)
        return c

    lax.fori_loop(0, NCHUNK, body, 0)
    plsc.subcore_barrier()
    pltpu.sync_copy(deg_sh.at[pl.ds(sid * RPT, RPT)],
                    deg_out.at[cid, pl.ds(sid * RPT, RPT)])


_sc_deg = functools.partial(
    pl.kernel,
    out_type=jax.ShapeDtypeStruct((NC, NP, DEGW), _f32),
    mesh=plsc.VectorSubcoreMesh(core_axis_name="c", subcore_axis_name="s",
                                num_cores=NC, num_subcores=NS),
    scratch_types=[
        pltpu.VMEM((NCHUNK, CHUNK), jnp.int32),
        pltpu.VMEM((CHUNK, DEGW), _f32),
        pltpu.VMEM((ZCH, DEGW), _f32),
        pltpu.VMEM_SHARED((NP, DEGW), _f32),
    ],
)(_deg_body)


def _agg_body(y, src3, dst3, out, idx_s, idx_d, rows, zbuf, acc_sh, sem, sem2):
    cid = lax.axis_index("c")
    sid = lax.axis_index("s")
    wid = cid * NS + sid
    _fill_rows(zbuf, ZCH, FF, 0.0)
    for t in range(RPT // ZCH):
        pltpu.sync_copy(zbuf, acc_sh.at[pl.ds(sid * RPT + t * ZCH, ZCH)])
    plsc.subcore_barrier()

    for boff, bsz in ((0, 64), (64, 56), (120, 5)):
        pltpu.sync_copy(src3.at[wid, pl.ds(boff, bsz)],
                        idx_s.at[pl.ds(0, bsz)])
        pltpu.sync_copy(dst3.at[wid, pl.ds(boff, bsz)],
                        idx_d.at[pl.ds(0, bsz)])

        def body(jj, c):
            j0 = jj * 2
            j1 = j0 + 1
            ga = pltpu.async_copy(y.at[idx_s.at[j0]], rows.at[0], sem)
            gb = pltpu.async_copy(y.at[idx_s.at[j1]], rows.at[1], sem2)
            ga.wait()
            pltpu.sync_copy(rows.at[0], acc_sh.at[idx_d.at[j0]], add=True)
            gb.wait()
            pltpu.sync_copy(rows.at[1], acc_sh.at[idx_d.at[j1]], add=True)
            return c

        lax.fori_loop(0, bsz // 2, body, 0)
        if bsz % 2:
            j = bsz - 1
            pltpu.async_copy(y.at[idx_s.at[j]], rows.at[0], sem).wait()
            pltpu.sync_copy(rows.at[0], acc_sh.at[idx_d.at[j]], add=True)
    plsc.subcore_barrier()
    pltpu.sync_copy(acc_sh.at[pl.ds(sid * RPT, RPT)],
                    out.at[cid, pl.ds(sid * RPT, RPT)])


_sc_agg = functools.partial(
    pl.kernel,
    out_type=jax.ShapeDtypeStruct((NC, NP, FF), _f32),
    mesh=plsc.VectorSubcoreMesh(core_axis_name="c", subcore_axis_name="s",
                                num_cores=NC, num_subcores=NS),
    scratch_types=[
        pltpu.VMEM((64, CHUNK), jnp.int32),
        pltpu.VMEM((64, CHUNK), jnp.int32),
        pltpu.VMEM((2, CHUNK, FF), _f32),
        pltpu.VMEM((ZCH, FF), _f32),
        pltpu.VMEM_SHARED((NP, FF), _f32),
        pltpu.SemaphoreType.DMA,
        pltpu.SemaphoreType.DMA,
    ],
)(_agg_body)


def _pre_body(x_ref, w_ref, deg_ref, y_ref, dinv_ref):
    deg = deg_ref[0, :NN, 0:1] + deg_ref[1, :NN, 0:1] + 1.0
    dinv = lax.rsqrt(deg)
    dinv_b = jnp.broadcast_to(dinv, (NN, FF))
    dinv_ref[...] = dinv_b
    xw = jnp.dot(x_ref[...], w_ref[...], preferred_element_type=_f32)
    y_ref[...] = xw * dinv_b


def _tc_pre(x, w1, deg):
    return pl.pallas_call(
        _pre_body,
        out_shape=(jax.ShapeDtypeStruct((NN, FF), _f32),
                   jax.ShapeDtypeStruct((NN, FF), _f32)),
    )(x, w1, deg)


def _layer_tail(prev, y, agg0, agg1, dinv, b, g, be):
    conv = dinv * (agg0 + agg1 + y) + b
    t = prev + conv
    m = jnp.mean(t, axis=0, keepdims=True)
    v = jnp.mean((t - m) * (t - m), axis=0, keepdims=True)
    xh = (t - m) * lax.rsqrt(v + 1e-5) * g + be
    neg = jnp.where(xh > 0, 0.0, xh)
    return jnp.where(xh > 0, xh, jnp.exp(neg) - 1.0)


def _mid_body(prev_ref, y_ref, agg_ref, dinv_ref, b_ref, g_ref, be_ref,
              wn_ref, hn_ref, yn_ref):
    dinv = dinv_ref[...]
    hn = _layer_tail(prev_ref[...], y_ref[...], agg_ref[0, :NN], agg_ref[1, :NN],
                     dinv, b_ref[...], g_ref[...], be_ref[...])
    hn_ref[...] = hn
    yn_ref[...] = jnp.dot(hn, wn_ref[...], preferred_element_type=_f32) * dinv


def _tc_mid(prev, y, agg, dinv, b, g, be, wn):
    return pl.pallas_call(
        _mid_body,
        out_shape=(jax.ShapeDtypeStruct((NN, FF), _f32),
                   jax.ShapeDtypeStruct((NN, FF), _f32)),
    )(prev, y, agg, dinv, b, g, be, wn)


def _final_body(prev_ref, y_ref, agg_ref, dinv_ref, b_ref, g_ref, be_ref,
                batch_ref, wr_ref, br_ref, out_ref):
    hn = _layer_tail(prev_ref[...], y_ref[...], agg_ref[0, :NN], agg_ref[1, :NN],
                     dinv_ref[...], b_ref[...], g_ref[...], be_ref[...])
    cols = lax.broadcasted_iota(jnp.int32, (NN, GG), 1)
    oh = (batch_ref[...] == cols).astype(_f32)
    dn = (((0,), (0,)), ((), ()))
    sums = lax.dot_general(oh, hn, dn, preferred_element_type=_f32)
    ones = jnp.ones((NN, 8), _f32)
    counts = lax.dot_general(oh, ones, dn, preferred_element_type=_f32)[:, 0:1]
    pooled = sums / jnp.maximum(counts, 1.0)
    out_ref[...] = (jnp.dot(pooled, wr_ref[...], preferred_element_type=_f32)
                    + br_ref[...])


def _tc_final(prev, y, agg, dinv, b, g, be, batch2d, wr, br):
    return pl.pallas_call(
        _final_body,
        out_shape=jax.ShapeDtypeStruct((GG, 2), _f32),
    )(prev, y, agg, dinv, b, g, be, batch2d, wr, br)


def kernel(x, edge_index, batch, W1, b1, g1, be1, W2, b2, g2, be2,
           W3, b3, g3, be3, Wr, br):
    src3 = edge_index[0].reshape(NW, NCHUNK, CHUNK)
    dst3 = edge_index[1].reshape(NW, NCHUNK, CHUNK)
    batch2d = batch.reshape(NN, 1)
    b1r, g1r, be1r = b1.reshape(1, FF), g1.reshape(1, FF), be1.reshape(1, FF)
    b2r, g2r, be2r = b2.reshape(1, FF), g2.reshape(1, FF), be2.reshape(1, FF)
    b3r, g3r, be3r = b3.reshape(1, FF), g3.reshape(1, FF), be3.reshape(1, FF)
    brr = br.reshape(1, 2)
    zeros = jnp.zeros((NN, FF), _f32)

    deg = _sc_deg(dst3)
    y1, dinv = _tc_pre(x, W1, deg)
    agg1 = _sc_agg(y1, src3, dst3)
    h1, y2 = _tc_mid(zeros, y1, agg1, dinv, b1r, g1r, be1r, W2)
    agg2 = _sc_agg(y2, src3, dst3)
    h2, y3 = _tc_mid(x, y2, agg2, dinv, b2r, g2r, be2r, W3)
    agg3 = _sc_agg(y3, src3, dst3)
    out = _tc_final(h1, y3, agg3, dinv, b3r, g3r, be3r, batch2d, Wr, brr)
    return out


# speedup vs baseline: 2.5532x; 1.0215x over previous
import functools

import jax
import jax.numpy as jnp
from jax import lax
from jax.experimental import pallas as pl
from jax.experimental.pallas import tpu as pltpu
from jax.experimental.pallas import tpu_sc as plsc

NN = 10000
EE = 320000
FF = 128
GG = 128
NC, NS = 2, 16
NW = NC * NS
CHUNK = 80
EPW = EE // NW
NCHUNK = EPW // CHUNK
NP = 10240
RPT = NP // NS
ZCH = 40
DEGW = 128

_f32 = jnp.float32


def _fill_rows(buf, nrows, width, value):
    vec = jnp.full((16,), value, _f32)

    def body(r, c):
        for k in range(width // 16):
            buf[r, pl.ds(k * 16, 16)] = vec
        return c

    lax.fori_loop(0, nrows, body, 0)


def _deg_body(dst3, deg_out, idx_v, ones_v, zbuf, deg_sh, dsem):
    cid = lax.axis_index("c")
    sid = lax.axis_index("s")
    wid = cid * NS + sid
    pltpu.sync_copy(dst3.at[wid], idx_v)
    _fill_rows(ones_v, CHUNK, DEGW, 1.0)
    _fill_rows(zbuf, ZCH, DEGW, 0.0)
    for t in range(RPT // ZCH):
        pltpu.sync_copy(zbuf, deg_sh.at[pl.ds(sid * RPT + t * ZCH, ZCH)])
    plsc.subcore_barrier()

    def body(jj, c):
        ds = [pltpu.async_copy(ones_v, deg_sh.at[idx_v.at[jj * 5 + k]],
                               dsem, add=True) for k in range(5)]
        for d in ds:
            d.wait()
        return c

    lax.fori_loop(0, NCHUNK // 5, body, 0)
    plsc.subcore_barrier()
    pltpu.sync_copy(deg_sh.at[pl.ds(sid * RPT, RPT)],
                    deg_out.at[cid, pl.ds(sid * RPT, RPT)])


_sc_deg = functools.partial(
    pl.kernel,
    out_type=jax.ShapeDtypeStruct((NC, NP, DEGW), _f32),
    mesh=plsc.VectorSubcoreMesh(core_axis_name="c", subcore_axis_name="s",
                                num_cores=NC, num_subcores=NS),
    scratch_types=[
        pltpu.VMEM((NCHUNK, CHUNK), jnp.int32),
        pltpu.VMEM((CHUNK, DEGW), _f32),
        pltpu.VMEM((ZCH, DEGW), _f32),
        pltpu.VMEM_SHARED((NP, DEGW), _f32),
        pltpu.SemaphoreType.DMA,
    ],
)(_deg_body)


def _agg_body(y, src3, dst3, out, idx_s, idx_d, rows, zbuf, acc_sh,
              sem, sem2, sem3, sem4):
    cid = lax.axis_index("c")
    sid = lax.axis_index("s")
    wid = cid * NS + sid
    _fill_rows(zbuf, ZCH, FF, 0.0)
    for t in range(RPT // ZCH):
        pltpu.sync_copy(zbuf, acc_sh.at[pl.ds(sid * RPT + t * ZCH, ZCH)])
    plsc.subcore_barrier()

    for boff, bsz in ((0, 64), (64, 56), (120, 5)):
        pltpu.sync_copy(src3.at[wid, pl.ds(boff, bsz)],
                        idx_s.at[pl.ds(0, bsz)])
        pltpu.sync_copy(dst3.at[wid, pl.ds(boff, bsz)],
                        idx_d.at[pl.ds(0, bsz)])

        def body(jj, c):
            j0 = jj * 2
            j1 = j0 + 1
            ga = pltpu.async_copy(y.at[idx_s.at[j0]], rows.at[0], sem)
            gb = pltpu.async_copy(y.at[idx_s.at[j1]], rows.at[1], sem2)
            ga.wait()
            sa = pltpu.async_copy(rows.at[0], acc_sh.at[idx_d.at[j0]], sem3,
                                  add=True)
            gb.wait()
            sb = pltpu.async_copy(rows.at[1], acc_sh.at[idx_d.at[j1]], sem4,
                                  add=True)
            sa.wait()
            sb.wait()
            return c

        lax.fori_loop(0, bsz // 2, body, 0)
        if bsz % 2:
            j = bsz - 1
            pltpu.async_copy(y.at[idx_s.at[j]], rows.at[0], sem).wait()
            pltpu.sync_copy(rows.at[0], acc_sh.at[idx_d.at[j]], add=True)
    plsc.subcore_barrier()
    pltpu.sync_copy(acc_sh.at[pl.ds(sid * RPT, RPT)],
                    out.at[cid, pl.ds(sid * RPT, RPT)])


_sc_agg = functools.partial(
    pl.kernel,
    out_type=jax.ShapeDtypeStruct((NC, NP, FF), _f32),
    mesh=plsc.VectorSubcoreMesh(core_axis_name="c", subcore_axis_name="s",
                                num_cores=NC, num_subcores=NS),
    scratch_types=[
        pltpu.VMEM((64, CHUNK), jnp.int32),
        pltpu.VMEM((64, CHUNK), jnp.int32),
        pltpu.VMEM((2, CHUNK, FF), _f32),
        pltpu.VMEM((ZCH, FF), _f32),
        pltpu.VMEM_SHARED((NP, FF), _f32),
        pltpu.SemaphoreType.DMA,
        pltpu.SemaphoreType.DMA,
        pltpu.SemaphoreType.DMA,
        pltpu.SemaphoreType.DMA,
    ],
)(_agg_body)


def _pre_body(x_ref, w_ref, deg_ref, y_ref, dinv_ref):
    deg = deg_ref[0, :NN, 0:1] + deg_ref[1, :NN, 0:1] + 1.0
    dinv = lax.rsqrt(deg)
    dinv_b = jnp.broadcast_to(dinv, (NN, FF))
    dinv_ref[...] = dinv_b
    xw = jnp.dot(x_ref[...], w_ref[...], preferred_element_type=_f32)
    y_ref[...] = xw * dinv_b


def _tc_pre(x, w1, deg):
    return pl.pallas_call(
        _pre_body,
        out_shape=(jax.ShapeDtypeStruct((NN, FF), _f32),
                   jax.ShapeDtypeStruct((NN, FF), _f32)),
    )(x, w1, deg)


def _layer_tail(prev, y, agg0, agg1, dinv, b, g, be):
    conv = dinv * (agg0 + agg1 + y) + b
    t = prev + conv
    m = jnp.mean(t, axis=0, keepdims=True)
    v = jnp.mean((t - m) * (t - m), axis=0, keepdims=True)
    xh = (t - m) * lax.rsqrt(v + 1e-5) * g + be
    neg = jnp.where(xh > 0, 0.0, xh)
    return jnp.where(xh > 0, xh, jnp.exp(neg) - 1.0)


def _mid_body(prev_ref, y_ref, agg_ref, dinv_ref, b_ref, g_ref, be_ref,
              wn_ref, hn_ref, yn_ref):
    dinv = dinv_ref[...]
    hn = _layer_tail(prev_ref[...], y_ref[...], agg_ref[0, :NN], agg_ref[1, :NN],
                     dinv, b_ref[...], g_ref[...], be_ref[...])
    hn_ref[...] = hn
    yn_ref[...] = jnp.dot(hn, wn_ref[...], preferred_element_type=_f32) * dinv


def _tc_mid(prev, y, agg, dinv, b, g, be, wn):
    return pl.pallas_call(
        _mid_body,
        out_shape=(jax.ShapeDtypeStruct((NN, FF), _f32),
                   jax.ShapeDtypeStruct((NN, FF), _f32)),
    )(prev, y, agg, dinv, b, g, be, wn)


def _final_body(prev_ref, y_ref, agg_ref, dinv_ref, b_ref, g_ref, be_ref,
                batch_ref, wr_ref, br_ref, out_ref):
    hn = _layer_tail(prev_ref[...], y_ref[...], agg_ref[0, :NN], agg_ref[1, :NN],
                     dinv_ref[...], b_ref[...], g_ref[...], be_ref[...])
    cols = lax.broadcasted_iota(jnp.int32, (NN, GG), 1)
    oh = (batch_ref[...] == cols).astype(_f32)
    dn = (((0,), (0,)), ((), ()))
    sums = lax.dot_general(oh, hn, dn, preferred_element_type=_f32)
    ones = jnp.ones((NN, 8), _f32)
    counts = lax.dot_general(oh, ones, dn, preferred_element_type=_f32)[:, 0:1]
    pooled = sums / jnp.maximum(counts, 1.0)
    out_ref[...] = (jnp.dot(pooled, wr_ref[...], preferred_element_type=_f32)
                    + br_ref[...])


def _tc_final(prev, y, agg, dinv, b, g, be, batch2d, wr, br):
    return pl.pallas_call(
        _final_body,
        out_shape=jax.ShapeDtypeStruct((GG, 2), _f32),
    )(prev, y, agg, dinv, b, g, be, batch2d, wr, br)


def kernel(x, edge_index, batch, W1, b1, g1, be1, W2, b2, g2, be2,
           W3, b3, g3, be3, Wr, br):
    src3 = edge_index[0].reshape(NW, NCHUNK, CHUNK)
    dst3 = edge_index[1].reshape(NW, NCHUNK, CHUNK)
    batch2d = batch.reshape(NN, 1)
    b1r, g1r, be1r = b1.reshape(1, FF), g1.reshape(1, FF), be1.reshape(1, FF)
    b2r, g2r, be2r = b2.reshape(1, FF), g2.reshape(1, FF), be2.reshape(1, FF)
    b3r, g3r, be3r = b3.reshape(1, FF), g3.reshape(1, FF), be3.reshape(1, FF)
    brr = br.reshape(1, 2)
    zeros = jnp.zeros((NN, FF), _f32)

    deg = _sc_deg(dst3)
    y1, dinv = _tc_pre(x, W1, deg)
    agg1 = _sc_agg(y1, src3, dst3)
    h1, y2 = _tc_mid(zeros, y1, agg1, dinv, b1r, g1r, be1r, W2)
    agg2 = _sc_agg(y2, src3, dst3)
    h2, y3 = _tc_mid(x, y2, agg2, dinv, b2r, g2r, be2r, W3)
    agg3 = _sc_agg(y3, src3, dst3)
    out = _tc_final(h1, y3, agg3, dinv, b3r, g3r, be3r, batch2d, Wr, brr)
    return out
